# baseline (speedup 1.0000x reference)
import jax
import jax.numpy as jnp
from jax import lax
from jax.experimental import pallas as pl
from jax.experimental.pallas import tpu as pltpu
from jax.experimental.pallas import tpu_sc as plsc

N = 100000
D = 128
H = 128
B = 512
NW = 32
C = 3128
LAST_W = NW - 1
LAST_ROWS = N - LAST_W * C
CHUNK = 128
NFULL = C // CHUNK
TAIL = C - NFULL * CHUNK
NFULL_LAST = LAST_ROWS // CHUNK
TAIL_LAST = LAST_ROWS - NFULL_LAST * CHUNK
EBUF = 3136
NQ = NFULL + 1

_mesh = plsc.VectorSubcoreMesh(core_axis_name="c", subcore_axis_name="s",
                               num_cores=2, num_subcores=16)
_sc_params = pltpu.CompilerParams(needs_layout_passes=False)


def _worker_meta():
    wid = (lax.axis_index("c") * 16 + lax.axis_index("s")).astype(jnp.int32)
    is_last = wid == LAST_W
    base = wid * C
    return wid, is_last, base


def _copy_seg(is_last, base, src_hbm, dst_v):
    @pl.when(jnp.logical_not(is_last))
    def _():
        pltpu.sync_copy(src_hbm.at[pl.ds(base, C)], dst_v.at[pl.ds(0, C)])

    @pl.when(is_last)
    def _():
        pltpu.sync_copy(src_hbm.at[pl.ds(base, LAST_ROWS)],
                        dst_v.at[pl.ds(0, LAST_ROWS)])


def _zero_buf(buf):
    def zrow(r, carry):
        for j in range(8):
            buf[r, pl.ds(j * 16, 16)] = jnp.zeros((16,), jnp.float32)
        return carry
    lax.fori_loop(0, CHUNK, zrow, 0)


def _init_shared_acc(x0, acc_sh, sid):
    _zero_buf(x0)

    @pl.when(sid == 0)
    def _():
        for g in range(B // CHUNK):
            pltpu.sync_copy(x0, acc_sh.at[pl.ds(g * CHUNK, CHUNK)])


def _loading_pass(x_hbm, base, is_last, x0, x1, sem_l0, sem_l1, process):
    nfull = jnp.where(is_last, NFULL_LAST, NFULL)

    def start_load(k, buf, sem):
        pltpu.async_copy(x_hbm.at[pl.ds(base + k * CHUNK, CHUNK)], buf, sem)

    def wait_load(buf, sem):
        pltpu.make_async_copy(x_hbm.at[pl.ds(base, CHUNK)], buf, sem).wait()

    start_load(0, x0, sem_l0)

    def pair(kk, carry):
        k0 = kk * 2

        @pl.when(k0 < nfull)
        def _():
            @pl.when(k0 + 1 < nfull)
            def _():
                start_load(k0 + 1, x1, sem_l1)
            wait_load(x0, sem_l0)
            process(x0, k0, CHUNK)

        @pl.when(k0 + 1 < nfull)
        def _():
            @pl.when(k0 + 2 < nfull)
            def _():
                start_load(k0 + 2, x0, sem_l0)
            wait_load(x1, sem_l1)
            process(x1, k0 + 1, CHUNK)
        return carry

    lax.fori_loop(0, (NFULL + 1) // 2, pair, 0)

    @pl.when(jnp.logical_not(is_last))
    def _():
        pltpu.sync_copy(x_hbm.at[pl.ds(base + NFULL * CHUNK, TAIL)],
                        x0.at[pl.ds(0, TAIL)])
        process(x0, NFULL, TAIL)

    @pl.when(is_last)
    def _():
        pltpu.sync_copy(x_hbm.at[pl.ds(base + NFULL_LAST * CHUNK, TAIL_LAST)],
                        x0.at[pl.ds(0, TAIL_LAST)])
        process(x0, NFULL_LAST, TAIL_LAST)


def _build_idx2d(idx2d, col0):
    for g in range(32):
        idx2d[g // 8, pl.ds((g % 8) * 16, 16)] = col0 + g * 16


def _zero_acc2d(acc2d):
    def zrow(r, carry):
        for rr in range(4):
            for j in range(8):
                acc2d[r * 4 + rr, pl.ds(j * 16, 16)] = jnp.zeros(
                    (16,), jnp.float32)
        return carry
    lax.fori_loop(0, B // 4, zrow, 0)


def _flush_streams(acc2d, acc_sh, idx2d, sem_s):
    for g in range(4):
        pltpu.async_copy(acc2d.at[pl.ds(g * CHUNK, CHUNK)],
                         acc_sh.at[idx2d.at[g]], sem_s, add=True)
    for g in range(4):
        pltpu.make_async_copy(acc2d.at[pl.ds(0, CHUNK)],
                              acc_sh.at[idx2d.at[0]], sem_s).wait()


_LANE0 = None


def _flush_run(acc2d, den_v, col0, sp16, acc_list, dr16, weighted):
    t16 = jnp.maximum(sp16, 0)
    for j in range(8):
        plsc.addupdate_scatter(acc2d, [t16, col0 + j * 16], acc_list[j])
    if weighted:
        plsc.addupdate_scatter(den_v, [t16 >> 7, t16 & 127], dr16,
                               mask=col0 == 0)


def _run_accum_process(xbuf, q, nrows, seg_v, e_v, spv_v, drv_v,
                       den_v, accrow_v, acc2d, col0, weighted):
    accs = [accrow_v[pl.ds(j * 16, 16)] for j in range(8)]
    sprev16 = spv_v[pl.ds(0, 16)]
    drun16 = drv_v[pl.ds(0, 16)] if weighted else jnp.zeros((16,),
                                                            jnp.float32)

    def row_step(rr, carry):
        sprev16, drun16, *accs = carry
        lr = q * CHUNK + rr
        idx = jnp.full((16,), lr, jnp.int32)
        seg16 = plsc.load_gather(seg_v, [idx])
        bnd16 = seg16 != sprev16
        bscal = lax.reduce_max(seg16 - sprev16, (0,)) != 0

        @pl.when(bscal)
        def _():
            _flush_run(acc2d, den_v, col0, sprev16, accs, drun16, weighted)

        if weighted:
            e16 = plsc.load_gather(e_v, [idx])
            xe = [xbuf[rr, pl.ds(j * 16, 16)] * e16 for j in range(8)]
            drun16 = jnp.where(bnd16, e16, drun16 + e16)
        else:
            xe = [xbuf[rr, pl.ds(j * 16, 16)] for j in range(8)]
        accs = [jnp.where(bnd16, xe[j], accs[j] + xe[j]) for j in range(8)]
        return (seg16, drun16, *accs)

    ru = 16 if nrows % 16 == 0 else 8

    def grp(g, carry):
        for r in range(ru):
            carry = row_step(g * ru + r, carry)
        return carry

    sprev16, drun16, *accs = lax.fori_loop(0, nrows // ru, grp,
                                           (sprev16, drun16, *accs))
    for j in range(8):
        accrow_v[pl.ds(j * 16, 16)] = accs[j]
    spv_v[pl.ds(0, 16)] = sprev16
    if weighted:
        drv_v[pl.ds(0, 16)] = drun16


def _sc_sum_body(x_hbm, seg_hbm, s_out,
                 x0, x1, seg_v, idx2d, accrow_v, spv_v, acc2d, acc_sh,
                 sem_l0, sem_l1, sem_s):
    wid, is_last, base = _worker_meta()
    cid = lax.axis_index("c")
    sid = lax.axis_index("s")
    col0 = lax.iota(jnp.int32, 16)
    _copy_seg(is_last, base, seg_hbm, seg_v)
    _init_shared_acc(x0, acc_sh, sid)
    _build_idx2d(idx2d, col0)
    _zero_acc2d(acc2d)
    for j in range(8):
        accrow_v[pl.ds(j * 16, 16)] = jnp.zeros((16,), jnp.float32)
    spv_v[pl.ds(0, 16)] = jnp.full((16,), -1, jnp.int32)
    plsc.subcore_barrier()

    def process(xbuf, q, nrows):
        _run_accum_process(xbuf, q, nrows, seg_v, None, spv_v, None,
                           None, accrow_v, acc2d, col0, weighted=False)

    _loading_pass(x_hbm, base, is_last, x0, x1, sem_l0, sem_l1, process)

    _flush_run(acc2d, None, col0, spv_v[pl.ds(0, 16)],
               [accrow_v[pl.ds(j * 16, 16)] for j in range(8)],
               None, False)

    _flush_streams(acc2d, acc_sh, idx2d, sem_s)
    plsc.subcore_barrier()

    @pl.when(sid == 0)
    def _():
        pltpu.sync_copy(acc_sh, s_out.at[cid])


_sc_sum0 = pl.kernel(
    _sc_sum_body,
    out_type=jax.ShapeDtypeStruct((2, B, D), jnp.float32),
    mesh=_mesh,
    scratch_types=[
        pltpu.VMEM((CHUNK, D), jnp.float32),
        pltpu.VMEM((CHUNK, D), jnp.float32),
        pltpu.VMEM((EBUF,), jnp.int32),
        pltpu.VMEM((4, 128), jnp.int32),
        pltpu.VMEM((D,), jnp.float32),
        pltpu.VMEM((16,), jnp.int32),
        pltpu.VMEM((B, D), jnp.float32),
        pltpu.VMEM_SHARED((B, D), jnp.float32),
        pltpu.SemaphoreType.DMA,
        pltpu.SemaphoreType.DMA,
        pltpu.SemaphoreType.DMA,
    ],
    compiler_params=_sc_params,
    name="sc_segsum0",
)


def _sc_att_body(x_hbm, seg_hbm, la_hbm, ra_hbm, s_out, d_out,
                 x0, x1, seg_v, la_v, e_v, ra_v, idx2d, accrow_v, spv_v,
                 drv_v, acc2d, den_v, acc_sh, sem_l0, sem_l1, sem_s):
    wid, is_last, base = _worker_meta()
    cid = lax.axis_index("c")
    sid = lax.axis_index("s")
    rows = jnp.where(is_last, LAST_ROWS, C).astype(jnp.int32)
    col0 = lax.iota(jnp.int32, 16)
    _copy_seg(is_last, base, seg_hbm, seg_v)
    _copy_seg(is_last, base, la_hbm, la_v)
    pltpu.sync_copy(ra_hbm, ra_v)
    _init_shared_acc(x0, acc_sh, sid)
    _build_idx2d(idx2d, col0)
    _zero_acc2d(acc2d)
    for j in range(8):
        accrow_v[pl.ds(j * 16, 16)] = jnp.zeros((16,), jnp.float32)
    spv_v[pl.ds(0, 16)] = jnp.full((16,), -1, jnp.int32)
    drv_v[pl.ds(0, 16)] = jnp.zeros((16,), jnp.float32)
    for g in range(4):
        for j in range(8):
            den_v[g, pl.ds(j * 16, 16)] = jnp.zeros((16,), jnp.float32)

    def egrp(g4, carry):
        for u in range(4):
            g = g4 * 4 + u
            valid = (g * 16 + col0) < rows
            seg16 = jnp.where(valid, seg_v[pl.ds(g * 16, 16)], 0)
            ra16 = plsc.load_gather(ra_v, [seg16])
            a = la_v[pl.ds(g * 16, 16)] + ra16
            a = jnp.where(a > 0, a, 0.01 * a)
            cshift = jnp.where(ra16 > 0, ra16, 0.01 * ra16)
            e_v[pl.ds(g * 16, 16)] = jnp.exp(a - cshift)
        return carry
    lax.fori_loop(0, EBUF // 64, egrp, 0)
    plsc.subcore_barrier()

    def process(xbuf, q, nrows):
        _run_accum_process(xbuf, q, nrows, seg_v, e_v, spv_v, drv_v,
                           den_v, accrow_v, acc2d, col0, weighted=True)

    _loading_pass(x_hbm, base, is_last, x0, x1, sem_l0, sem_l1, process)

    _flush_run(acc2d, den_v, col0, spv_v[pl.ds(0, 16)],
               [accrow_v[pl.ds(j * 16, 16)] for j in range(8)],
               drv_v[pl.ds(0, 16)], True)

    _flush_streams(acc2d, acc_sh, idx2d, sem_s)
    plsc.subcore_barrier()

    @pl.when(sid == 0)
    def _():
        pltpu.sync_copy(acc_sh, s_out.at[cid])
    pltpu.sync_copy(den_v, d_out.at[pl.ds(wid * 4, 4)])


_sc_att = pl.kernel(
    _sc_att_body,
    out_type=[jax.ShapeDtypeStruct((2, B, D), jnp.float32),
              jax.ShapeDtypeStruct((NW * 4, 128), jnp.float32)],
    mesh=_mesh,
    scratch_types=[
        pltpu.VMEM((CHUNK, D), jnp.float32),
        pltpu.VMEM((CHUNK, D), jnp.float32),
        pltpu.VMEM((EBUF,), jnp.int32),
        pltpu.VMEM((EBUF,), jnp.float32),
        pltpu.VMEM((EBUF,), jnp.float32),
        pltpu.VMEM((B,), jnp.float32),
        pltpu.VMEM((4, 128), jnp.int32),
        pltpu.VMEM((D,), jnp.float32),
        pltpu.VMEM((16,), jnp.int32),
        pltpu.VMEM((16,), jnp.float32),
        pltpu.VMEM((B, D), jnp.float32),
        pltpu.VMEM((4, 128), jnp.float32),
        pltpu.VMEM_SHARED((B, D), jnp.float32),
        pltpu.SemaphoreType.DMA,
        pltpu.SemaphoreType.DMA,
        pltpu.SemaphoreType.DMA,
    ],
    compiler_params=_sc_params,
    name="sc_att_segsum",
)


def _tc_la_body(x_ref, w_ref, o_ref):
    o_ref[0] = lax.dot_general(w_ref[...], x_ref[...],
                               (((1,), (1,)), ((), ())),
                               preferred_element_type=jnp.float32)


_tc_la = pl.pallas_call(
    _tc_la_body,
    grid=(50,),
    in_specs=[pl.BlockSpec((2000, D), lambda k: (k, 0)),
              pl.BlockSpec((1, D), lambda k: (0, 0))],
    out_specs=pl.BlockSpec((1, 1, 2000), lambda k: (k, 0, 0)),
    out_shape=jax.ShapeDtypeStruct((50, 1, 2000), jnp.float32),
)


def _tc_comb0_body(sp_ref, war_ref, out0_ref, ra_ref):
    out0 = jnp.sum(sp_ref[...], axis=0)
    out0_ref[...] = out0
    ra_ref[...] = lax.dot_general(war_ref[...], out0,
                                  (((1,), (1,)), ((), ())),
                                  preferred_element_type=jnp.float32)


_tc_comb0 = pl.pallas_call(
    _tc_comb0_body,
    out_shape=[jax.ShapeDtypeStruct((B, D), jnp.float32),
               jax.ShapeDtypeStruct((1, B), jnp.float32)],
)


def _mmT(a, w):
    return lax.dot_general(a, w, (((1,), (1,)), ((), ())),
                           preferred_element_type=jnp.float32)


def _gru_silu(pooled, prev, Wn, Wih, Whh, bih, bhh):
    sn = _mmT(pooled, Wn)
    h = jnp.where(sn > 0, sn, jnp.exp(sn) - 1.0)
    gi = _mmT(h, Wih) + bih
    gh = _mmT(prev, Whh) + bhh
    r = jax.nn.sigmoid(gi[:, 0:H] + gh[:, 0:H])
    z = jax.nn.sigmoid(gi[:, H:2 * H] + gh[:, H:2 * H])
    n = jnp.tanh(gi[:, 2 * H:] + r * gh[:, 2 * H:])
    g = (1.0 - z) * n + z * prev
    return g * jax.nn.sigmoid(g)


def _tc_iter_body(sp_ref, inv_ref, prev_ref, Wn_ref, Wih_ref, Whh_ref,
                  bih_ref, bhh_ref, war_ref, out_ref, ra_ref):
    s = jnp.sum(sp_ref[...], axis=0)
    pooled = s * inv_ref[...]
    out = _gru_silu(pooled, prev_ref[...], Wn_ref[...], Wih_ref[...],
                    Whh_ref[...], bih_ref[...], bhh_ref[...])
    out_ref[...] = out
    ra_ref[...] = lax.dot_general(war_ref[...], out,
                                  (((1,), (1,)), ((), ())),
                                  preferred_element_type=jnp.float32)


_tc_iter = pl.pallas_call(
    _tc_iter_body,
    out_shape=[jax.ShapeDtypeStruct((B, D), jnp.float32),
               jax.ShapeDtypeStruct((1, B), jnp.float32)],
)


def _tc_final_body(sp_ref, inv_ref, prev_ref, Wn_ref, Wih_ref, Whh_ref,
                   bih_ref, bhh_ref, Wl_ref, bl_ref, y_ref):
    s = jnp.sum(sp_ref[...], axis=0)
    pooled = s * inv_ref[...]
    out = _gru_silu(pooled, prev_ref[...], Wn_ref[...], Wih_ref[...],
                    Whh_ref[...], bih_ref[...], bhh_ref[...])
    y_ref[...] = _mmT(out, Wl_ref[...]) + bl_ref[...]


_tc_final = pl.pallas_call(
    _tc_final_body,
    out_shape=jax.ShapeDtypeStruct((B, D), jnp.float32),
)


def kernel(x, segment_ids, w_att_l, w_att_r, W_node, W_ih, W_hh,
           b_ih, b_hh, W_lin, b_lin):
    seg = segment_ids.astype(jnp.int32)
    war = w_att_r.reshape(1, D)
    bih = b_ih.reshape(1, 3 * H)
    bhh = b_hh.reshape(1, 3 * H)
    bl = b_lin.reshape(1, D)

    la = _tc_la(x, w_att_l.reshape(1, D)).reshape(N)
    s0p = _sc_sum0(x, seg)
    out0, ra = _tc_comb0(s0p, war)

    prev = out0
    ra_flat = ra.reshape(B)
    for t in range(2):
        sp, dp = _sc_att(x, seg, la, ra_flat)
        den = jnp.sum(dp.reshape(NW, B), axis=0)
        inv = (1.0 / jnp.where(den == 0, 1.0, den)).reshape(B, 1)
        if t == 0:
            prev, ra = _tc_iter(sp, inv, prev, W_node, W_ih, W_hh,
                                bih, bhh, war)
            ra_flat = ra.reshape(B)
        else:
            y = _tc_final(sp, inv, prev, W_node, W_ih, W_hh,
                          bih, bhh, W_lin, bl)
    return y


# speedup vs baseline: 1.6204x; 1.6204x over previous
import jax
import jax.numpy as jnp
from jax import lax
from jax.experimental import pallas as pl
from jax.experimental.pallas import tpu as pltpu
from jax.experimental.pallas import tpu_sc as plsc

N = 100000
D = 128
H = 128
B = 512
NW = 32
C = 3128
LAST_W = NW - 1
LAST_ROWS = N - LAST_W * C
CHUNK = 128
NFULL = C // CHUNK
TAIL = C - NFULL * CHUNK
NFULL_LAST = LAST_ROWS // CHUNK
TAIL_LAST = LAST_ROWS - NFULL_LAST * CHUNK
EBUF = 3136
NQ = NFULL + 1

_mesh = plsc.VectorSubcoreMesh(core_axis_name="c", subcore_axis_name="s",
                               num_cores=2, num_subcores=16)
_sc_params = pltpu.CompilerParams(needs_layout_passes=False)


def _worker_meta():
    wid = (lax.axis_index("c") * 16 + lax.axis_index("s")).astype(jnp.int32)
    is_last = wid == LAST_W
    base = wid * C
    return wid, is_last, base


def _copy_seg(is_last, base, src_hbm, dst_v):
    @pl.when(jnp.logical_not(is_last))
    def _():
        pltpu.sync_copy(src_hbm.at[pl.ds(base, C)], dst_v.at[pl.ds(0, C)])

    @pl.when(is_last)
    def _():
        pltpu.sync_copy(src_hbm.at[pl.ds(base, LAST_ROWS)],
                        dst_v.at[pl.ds(0, LAST_ROWS)])


def _zero_buf(buf):
    def zrow(r, carry):
        for j in range(8):
            buf[r, pl.ds(j * 16, 16)] = jnp.zeros((16,), jnp.float32)
        return carry
    lax.fori_loop(0, CHUNK, zrow, 0)


def _init_shared_acc(x0, acc_sh, sid):
    _zero_buf(x0)

    @pl.when(sid == 0)
    def _():
        for g in range(B // CHUNK):
            pltpu.sync_copy(x0, acc_sh.at[pl.ds(g * CHUNK, CHUNK)])


def _streaming_pass(x_hbm, seg_hbm, seg2d_v, acc_sh, base, is_last,
                    x0, x1, m0, m1, t56, t88,
                    sem_l0, sem_l1, sem_s0, sem_s1, compute,
                    aliased=False):
    nfull = jnp.where(is_last, NFULL_LAST, NFULL)

    def start_load(k, buf, sem):
        pltpu.async_copy(x_hbm.at[pl.ds(base + k * CHUNK, CHUNK)], buf, sem)

    def wait_load(buf, sem):
        pltpu.make_async_copy(x_hbm.at[pl.ds(base, CHUNK)], buf, sem).wait()

    def start_stream(buf, k, sem):
        pltpu.async_copy(buf, acc_sh.at[seg2d_v.at[k]], sem, add=True)

    def wait_stream(buf, sem):
        pltpu.make_async_copy(buf, acc_sh.at[seg2d_v.at[0]], sem).wait()

    start_load(0, x0, sem_l0)

    def half(k, xbuf, mbuf, sem_l, sem_s, load_k, xload, sem_lo, sem_so):
        @pl.when(load_k < nfull)
        def _():
            if aliased:
                @pl.when(load_k >= 2)
                def _():
                    wait_stream(xload, sem_so)
            start_load(load_k, xload, sem_lo)

        @pl.when(jnp.logical_and(k >= 2, jnp.logical_not(aliased)))
        def _():
            wait_stream(mbuf, sem_s)
        wait_load(xbuf, sem_l)
        compute(xbuf, mbuf, k, CHUNK)
        start_stream(mbuf, k, sem_s)

    def pair(kk, carry):
        k0 = kk * 2

        @pl.when(k0 < nfull)
        def _():
            half(k0, x0, m0, sem_l0, sem_s0, k0 + 1, x1, sem_l1, sem_s1)

        @pl.when(k0 + 1 < nfull)
        def _():
            half(k0 + 1, x1, m1, sem_l1, sem_s1, k0 + 2, x0, sem_l0, sem_s0)
        return carry

    lax.fori_loop(0, (NFULL + 1) // 2, pair, 0)
    wait_stream(m0, sem_s0)
    wait_stream(m1, sem_s1)

    @pl.when(jnp.logical_not(is_last))
    def _():
        pltpu.sync_copy(seg_hbm.at[pl.ds(base + NFULL * CHUNK, TAIL)], t56)
        pltpu.sync_copy(x_hbm.at[pl.ds(base + NFULL * CHUNK, TAIL)],
                        x0.at[pl.ds(0, TAIL)])
        compute(x0, m0, NFULL, TAIL)
        pltpu.async_copy(m0.at[pl.ds(0, TAIL)], acc_sh.at[t56], sem_s0,
                         add=True).wait()

    @pl.when(is_last)
    def _():
        pltpu.sync_copy(seg_hbm.at[pl.ds(base + NFULL_LAST * CHUNK,
                                         TAIL_LAST)], t88)
        pltpu.sync_copy(x_hbm.at[pl.ds(base + NFULL_LAST * CHUNK, TAIL_LAST)],
                        x0.at[pl.ds(0, TAIL_LAST)])
        compute(x0, m0, NFULL_LAST, TAIL_LAST)
        pltpu.async_copy(m0.at[pl.ds(0, TAIL_LAST)], acc_sh.at[t88], sem_s0,
                         add=True).wait()


def _sc_sum_body(x_hbm, seg_hbm, s_out,
                 x0, x1, seg_v, seg2d_v, t56, t88, acc_sh,
                 sem_l0, sem_l1, sem_s0, sem_s1):
    wid, is_last, base = _worker_meta()
    cid = lax.axis_index("c")
    sid = lax.axis_index("s")
    _copy_seg(is_last, base, seg_hbm, seg_v)
    _init_shared_acc(x0, acc_sh, sid)

    def sgrp(g4, carry):
        for u in range(4):
            g = g4 * 4 + u
            seg2d_v[g // 8, pl.ds((g % 8) * 16, 16)] = \
                seg_v[pl.ds(g * 16, 16)]
        return carry
    lax.fori_loop(0, (NFULL * CHUNK) // 64, sgrp, 0)
    plsc.subcore_barrier()

    def compute(xbuf, mbuf, q, nrows):
        pass

    _streaming_pass(x_hbm, seg_hbm, seg2d_v, acc_sh, base, is_last,
                    x0, x1, x0, x1, t56, t88,
                    sem_l0, sem_l1, sem_s0, sem_s1, compute, aliased=True)
    plsc.subcore_barrier()

    @pl.when(sid == 0)
    def _():
        pltpu.sync_copy(acc_sh, s_out.at[cid])


_sc_sum0 = pl.kernel(
    _sc_sum_body,
    out_type=jax.ShapeDtypeStruct((2, B, D), jnp.float32),
    mesh=_mesh,
    scratch_types=[
        pltpu.VMEM((CHUNK, D), jnp.float32),
        pltpu.VMEM((CHUNK, D), jnp.float32),
        pltpu.VMEM((EBUF,), jnp.int32),
        pltpu.VMEM((NQ, 128), jnp.int32),
        pltpu.VMEM((TAIL,), jnp.int32),
        pltpu.VMEM((TAIL_LAST,), jnp.int32),
        pltpu.VMEM_SHARED((B, D), jnp.float32),
        pltpu.SemaphoreType.DMA,
        pltpu.SemaphoreType.DMA,
        pltpu.SemaphoreType.DMA,
        pltpu.SemaphoreType.DMA,
    ],
    compiler_params=_sc_params,
    name="sc_segsum0",
)


def _sc_att_body(x_hbm, seg_hbm, la_hbm, ra_hbm, s_out, d_out,
                 x0, x1, m0, m1, seg_v, la_v, e_v, seg2d_v, ra_v, den_v,
                 t56, t88, acc_sh, sem_l0, sem_l1, sem_s0, sem_s1):
    wid, is_last, base = _worker_meta()
    cid = lax.axis_index("c")
    sid = lax.axis_index("s")
    rows = jnp.where(is_last, LAST_ROWS, C).astype(jnp.int32)
    col0 = lax.iota(jnp.int32, 16)
    _copy_seg(is_last, base, seg_hbm, seg_v)
    _copy_seg(is_last, base, la_hbm, la_v)
    pltpu.sync_copy(ra_hbm, ra_v)
    _init_shared_acc(x0, acc_sh, sid)
    for g in range(4):
        for j in range(8):
            den_v[g, pl.ds(j * 16, 16)] = jnp.zeros((16,), jnp.float32)

    def egrp(g4, carry):
        for u in range(4):
            g = g4 * 4 + u
            valid = (g * 16 + col0) < rows
            seg16 = jnp.where(valid, seg_v[pl.ds(g * 16, 16)], 0)
            seg2d_v[g // 8, pl.ds((g % 8) * 16, 16)] = seg16
            ra16 = plsc.load_gather(ra_v, [seg16])
            a = la_v[pl.ds(g * 16, 16)] + ra16
            a = jnp.where(a > 0, a, 0.01 * a)
            cshift = jnp.where(ra16 > 0, ra16, 0.01 * ra16)
            e_v[pl.ds(g * 16, 16)] = jnp.exp(a - cshift)
        return carry
    lax.fori_loop(0, EBUF // 64, egrp, 0)
    plsc.subcore_barrier()

    lane0 = col0 == 0

    def compute(xbuf, mbuf, q, nrows):
        def row_body(r):
            lr = q * CHUNK + r
            idx = jnp.full((16,), lr, jnp.int32)
            seg16 = plsc.load_gather(seg_v, [idx])
            e16 = plsc.load_gather(e_v, [idx])
            plsc.addupdate_scatter(den_v, [seg16 >> 7, seg16 & 127], e16,
                                   mask=lane0)
            for j in range(8):
                mbuf[r, pl.ds(j * 16, 16)] = xbuf[r, pl.ds(j * 16, 16)] * e16

        if nrows % 32 == 0:
            def grp(g, carry):
                for r in range(32):
                    row_body(g * 32 + r)
                return carry
            lax.fori_loop(0, nrows // 32, grp, 0)
        else:
            def grp(g, carry):
                for r in range(8):
                    row_body(g * 8 + r)
                return carry
            lax.fori_loop(0, nrows // 8, grp, 0)

    _streaming_pass(x_hbm, seg_hbm, seg2d_v, acc_sh, base, is_last,
                    x0, x1, m0, m1, t56, t88,
                    sem_l0, sem_l1, sem_s0, sem_s1, compute)
    plsc.subcore_barrier()

    @pl.when(sid == 0)
    def _():
        pltpu.sync_copy(acc_sh, s_out.at[cid])
    pltpu.sync_copy(den_v, d_out.at[pl.ds(wid * 4, 4)])


_sc_att = pl.kernel(
    _sc_att_body,
    out_type=[jax.ShapeDtypeStruct((2, B, D), jnp.float32),
              jax.ShapeDtypeStruct((NW * 4, 128), jnp.float32)],
    mesh=_mesh,
    scratch_types=[
        pltpu.VMEM((CHUNK, D), jnp.float32),
        pltpu.VMEM((CHUNK, D), jnp.float32),
        pltpu.VMEM((CHUNK, D), jnp.float32),
        pltpu.VMEM((CHUNK, D), jnp.float32),
        pltpu.VMEM((EBUF,), jnp.int32),
        pltpu.VMEM((EBUF,), jnp.float32),
        pltpu.VMEM((EBUF,), jnp.float32),
        pltpu.VMEM((NQ, 128), jnp.int32),
        pltpu.VMEM((B,), jnp.float32),
        pltpu.VMEM((4, 128), jnp.float32),
        pltpu.VMEM((TAIL,), jnp.int32),
        pltpu.VMEM((TAIL_LAST,), jnp.int32),
        pltpu.VMEM_SHARED((B, D), jnp.float32),
        pltpu.SemaphoreType.DMA,
        pltpu.SemaphoreType.DMA,
        pltpu.SemaphoreType.DMA,
        pltpu.SemaphoreType.DMA,
    ],
    compiler_params=_sc_params,
    name="sc_att_segsum",
)


def _tc_la_body(x_ref, w_ref, o_ref):
    o_ref[0] = lax.dot_general(w_ref[...], x_ref[...],
                               (((1,), (1,)), ((), ())),
                               preferred_element_type=jnp.float32)


_tc_la = pl.pallas_call(
    _tc_la_body,
    grid=(50,),
    in_specs=[pl.BlockSpec((2000, D), lambda k: (k, 0)),
              pl.BlockSpec((1, D), lambda k: (0, 0))],
    out_specs=pl.BlockSpec((1, 1, 2000), lambda k: (k, 0, 0)),
    out_shape=jax.ShapeDtypeStruct((50, 1, 2000), jnp.float32),
)


def _tc_comb0_body(sp_ref, war_ref, out0_ref, ra_ref):
    out0 = jnp.sum(sp_ref[...], axis=0)
    out0_ref[...] = out0
    ra_ref[...] = lax.dot_general(war_ref[...], out0,
                                  (((1,), (1,)), ((), ())),
                                  preferred_element_type=jnp.float32)


_tc_comb0 = pl.pallas_call(
    _tc_comb0_body,
    out_shape=[jax.ShapeDtypeStruct((B, D), jnp.float32),
               jax.ShapeDtypeStruct((1, B), jnp.float32)],
)


def _mmT(a, w):
    return lax.dot_general(a, w, (((1,), (1,)), ((), ())),
                           preferred_element_type=jnp.float32)


def _gru_silu(pooled, prev, Wn, Wih, Whh, bih, bhh):
    sn = _mmT(pooled, Wn)
    h = jnp.where(sn > 0, sn, jnp.exp(sn) - 1.0)
    gi = _mmT(h, Wih) + bih
    gh = _mmT(prev, Whh) + bhh
    r = jax.nn.sigmoid(gi[:, 0:H] + gh[:, 0:H])
    z = jax.nn.sigmoid(gi[:, H:2 * H] + gh[:, H:2 * H])
    n = jnp.tanh(gi[:, 2 * H:] + r * gh[:, 2 * H:])
    g = (1.0 - z) * n + z * prev
    return g * jax.nn.sigmoid(g)


def _tc_iter_body(sp_ref, inv_ref, prev_ref, Wn_ref, Wih_ref, Whh_ref,
                  bih_ref, bhh_ref, war_ref, out_ref, ra_ref):
    s = jnp.sum(sp_ref[...], axis=0)
    pooled = s * inv_ref[...]
    out = _gru_silu(pooled, prev_ref[...], Wn_ref[...], Wih_ref[...],
                    Whh_ref[...], bih_ref[...], bhh_ref[...])
    out_ref[...] = out
    ra_ref[...] = lax.dot_general(war_ref[...], out,
                                  (((1,), (1,)), ((), ())),
                                  preferred_element_type=jnp.float32)


_tc_iter = pl.pallas_call(
    _tc_iter_body,
    out_shape=[jax.ShapeDtypeStruct((B, D), jnp.float32),
               jax.ShapeDtypeStruct((1, B), jnp.float32)],
)


def _tc_final_body(sp_ref, inv_ref, prev_ref, Wn_ref, Wih_ref, Whh_ref,
                   bih_ref, bhh_ref, Wl_ref, bl_ref, y_ref):
    s = jnp.sum(sp_ref[...], axis=0)
    pooled = s * inv_ref[...]
    out = _gru_silu(pooled, prev_ref[...], Wn_ref[...], Wih_ref[...],
                    Whh_ref[...], bih_ref[...], bhh_ref[...])
    y_ref[...] = _mmT(out, Wl_ref[...]) + bl_ref[...]


_tc_final = pl.pallas_call(
    _tc_final_body,
    out_shape=jax.ShapeDtypeStruct((B, D), jnp.float32),
)


def kernel(x, segment_ids, w_att_l, w_att_r, W_node, W_ih, W_hh,
           b_ih, b_hh, W_lin, b_lin):
    seg = segment_ids.astype(jnp.int32)
    war = w_att_r.reshape(1, D)
    bih = b_ih.reshape(1, 3 * H)
    bhh = b_hh.reshape(1, 3 * H)
    bl = b_lin.reshape(1, D)

    la = _tc_la(x, w_att_l.reshape(1, D)).reshape(N)
    s0p = _sc_sum0(x, seg)
    out0, ra = _tc_comb0(s0p, war)

    prev = out0
    ra_flat = ra.reshape(B)
    for t in range(2):
        sp, dp = _sc_att(x, seg, la, ra_flat)
        den = jnp.sum(dp.reshape(NW, B), axis=0)
        inv = (1.0 / jnp.where(den == 0, 1.0, den)).reshape(B, 1)
        if t == 0:
            prev, ra = _tc_iter(sp, inv, prev, W_node, W_ih, W_hh,
                                bih, bhh, war)
            ra_flat = ra.reshape(B)
        else:
            y = _tc_final(sp, inv, prev, W_node, W_ih, W_hh,
                          bih, bhh, W_lin, bl)
    return y
